# baseline (device time: 29877 ns/iter reference)
import jax
import jax.numpy as jnp
from jax import lax
from jax.experimental import pallas as pl
from jax.experimental.pallas import tpu as pltpu

N_DEV = 16


def kernel(x):
    m, n_total = x.shape
    assert n_total == N_DEV * m

    def body(x_hbm, out_hbm, x_vm, send_buf, recv_buf, out_stage,
             in_sem, out_sems, send_sems, recv_sems):
        me = lax.axis_index("i")

        fetch = pltpu.make_async_copy(x_hbm, x_vm, in_sem)
        fetch.start()
        own = pltpu.make_async_copy(
            x_hbm.at[:, pl.ds(me * m, m)],
            out_hbm.at[pl.ds(me * m, m), :],
            out_sems.at[me],
        )
        own.start()

        barrier_sem = pltpu.get_barrier_semaphore()
        for j in range(N_DEV):
            @pl.when(j != me)
            def _():
                pl.semaphore_signal(
                    barrier_sem, inc=1,
                    device_id=(j,), device_id_type=pl.DeviceIdType.MESH,
                )
        pl.semaphore_wait(barrier_sem, N_DEV - 1)
        fetch.wait()

        sends = []
        for k in range(1, N_DEV):
            dst = lax.rem(me + k, N_DEV)
            send_buf[dst, :, :] = x_vm[:, pl.ds(dst * m, m)].astype(jnp.bfloat16)
            rdma = pltpu.make_async_remote_copy(
                src_ref=send_buf.at[dst],
                dst_ref=recv_buf.at[me],
                send_sem=send_sems.at[dst],
                recv_sem=recv_sems.at[me],
                device_id=(dst,),
                device_id_type=pl.DeviceIdType.MESH,
            )
            rdma.start()
            sends.append(rdma)

        out_copies = [own]
        for k in range(1, N_DEV):
            src = lax.rem(me - k + N_DEV, N_DEV)
            recv = pltpu.make_async_remote_copy(
                src_ref=send_buf.at[src],
                dst_ref=recv_buf.at[src],
                send_sem=send_sems.at[src],
                recv_sem=recv_sems.at[src],
                device_id=(src,),
                device_id_type=pl.DeviceIdType.MESH,
            )
            recv.wait_recv()
            out_stage[src, :, :] = recv_buf[src].astype(jnp.float32)
            store = pltpu.make_async_copy(
                out_stage.at[src],
                out_hbm.at[pl.ds(src * m, m), :],
                out_sems.at[src],
            )
            store.start()
            out_copies.append(store)

        for rdma in sends:
            rdma.wait_send()
        for cp in out_copies:
            cp.wait()

    out_shape = jax.ShapeDtypeStruct((n_total, m), jnp.float32)
    return pl.pallas_call(
        body,
        out_shape=out_shape,
        in_specs=[pl.BlockSpec(memory_space=pl.ANY)],
        out_specs=pl.BlockSpec(memory_space=pl.ANY),
        scratch_shapes=[
            pltpu.VMEM((m, n_total), jnp.float32),
            pltpu.VMEM((N_DEV, m, m), jnp.bfloat16),
            pltpu.VMEM((N_DEV, m, m), jnp.bfloat16),
            pltpu.VMEM((N_DEV, m, m), jnp.float32),
            pltpu.SemaphoreType.DMA,
            pltpu.SemaphoreType.DMA((N_DEV,)),
            pltpu.SemaphoreType.DMA((N_DEV,)),
            pltpu.SemaphoreType.DMA((N_DEV,)),
        ],
        compiler_params=pltpu.CompilerParams(collective_id=0),
    )(x)


# device time: 29514 ns/iter; 1.0123x vs baseline; 1.0123x over previous
import jax
import jax.numpy as jnp
from jax import lax
from jax.experimental import pallas as pl
from jax.experimental.pallas import tpu as pltpu

N_DEV = 16


def kernel(x):
    m, n_total = x.shape
    assert n_total == N_DEV * m

    def body(x_hbm, out_hbm, x_vm, send_buf, recv_buf, out_stage,
             in_sem, out_sems, send_sems, recv_sems):
        me = lax.axis_index("i")

        barrier_sem = pltpu.get_barrier_semaphore()
        for j in range(N_DEV):
            @pl.when(j != me)
            def _():
                pl.semaphore_signal(
                    barrier_sem, inc=1,
                    device_id=(j,), device_id_type=pl.DeviceIdType.MESH,
                )

        fetch = pltpu.make_async_copy(x_hbm, x_vm, in_sem)
        fetch.start()
        own = pltpu.make_async_copy(
            x_hbm.at[:, pl.ds(me * m, m)],
            out_hbm.at[pl.ds(me * m, m), :],
            out_sems.at[me],
        )
        own.start()
        fetch.wait()

        for k in range(1, N_DEV):
            dst = lax.rem(me + k, N_DEV)
            send_buf[dst, :, :] = x_vm[:, pl.ds(dst * m, m)].astype(jnp.bfloat16)

        pl.semaphore_wait(barrier_sem, N_DEV - 1)

        sends = []
        for k in range(1, N_DEV):
            dst = lax.rem(me + k, N_DEV)
            rdma = pltpu.make_async_remote_copy(
                src_ref=send_buf.at[dst],
                dst_ref=recv_buf.at[me],
                send_sem=send_sems.at[dst],
                recv_sem=recv_sems.at[me],
                device_id=(dst,),
                device_id_type=pl.DeviceIdType.MESH,
            )
            rdma.start()
            sends.append(rdma)

        out_copies = [own]
        for k in range(1, N_DEV):
            src = lax.rem(me - k + N_DEV, N_DEV)
            recv = pltpu.make_async_remote_copy(
                src_ref=send_buf.at[src],
                dst_ref=recv_buf.at[src],
                send_sem=send_sems.at[src],
                recv_sem=recv_sems.at[src],
                device_id=(src,),
                device_id_type=pl.DeviceIdType.MESH,
            )
            recv.wait_recv()
            out_stage[src, :, :] = recv_buf[src].astype(jnp.float32)
            store = pltpu.make_async_copy(
                out_stage.at[src],
                out_hbm.at[pl.ds(src * m, m), :],
                out_sems.at[src],
            )
            store.start()
            out_copies.append(store)

        for rdma in sends:
            rdma.wait_send()
        for cp in out_copies:
            cp.wait()

    out_shape = jax.ShapeDtypeStruct((n_total, m), jnp.float32)
    return pl.pallas_call(
        body,
        out_shape=out_shape,
        in_specs=[pl.BlockSpec(memory_space=pl.ANY)],
        out_specs=pl.BlockSpec(memory_space=pl.ANY),
        scratch_shapes=[
            pltpu.VMEM((m, n_total), jnp.float32),
            pltpu.VMEM((N_DEV, m, m), jnp.bfloat16),
            pltpu.VMEM((N_DEV, m, m), jnp.bfloat16),
            pltpu.VMEM((N_DEV, m, m), jnp.float32),
            pltpu.SemaphoreType.DMA,
            pltpu.SemaphoreType.DMA((N_DEV,)),
            pltpu.SemaphoreType.DMA((N_DEV,)),
            pltpu.SemaphoreType.DMA((N_DEV,)),
        ],
        compiler_params=pltpu.CompilerParams(collective_id=0),
    )(x)
